# baseline (speedup 1.0000x reference)
import functools

import jax
import jax.numpy as jnp
from jax import lax
from jax.experimental import pallas as pl
from jax.experimental.pallas import tpu as pltpu
from jax.experimental.pallas import tpu_sc as plsc

_LANES = 16
_NC = 2
_NS = 16


@functools.lru_cache(maxsize=None)
def _make_sc_add(B, S, D):
    NW = _NC * _NS
    s_per_w = S // NW
    SCHUNK = 16
    n_iter = s_per_w // SCHUNK

    mesh = plsc.VectorSubcoreMesh(core_axis_name="c", subcore_axis_name="s")

    @functools.partial(
        pl.kernel,
        out_type=jax.ShapeDtypeStruct((B, S, D), jnp.float32),
        mesh=mesh,
        scratch_types=[
            pltpu.VMEM((B, SCHUNK, D), jnp.float32),
            pltpu.VMEM((B, SCHUNK, D), jnp.float32),
            pltpu.VMEM((SCHUNK, D), jnp.float32),
            pltpu.VMEM((SCHUNK, D), jnp.float32),
            pltpu.SemaphoreType.DMA,
            pltpu.SemaphoreType.DMA,
            pltpu.SemaphoreType.DMA,
            pltpu.SemaphoreType.DMA,
            pltpu.SemaphoreType.DMA,
            pltpu.SemaphoreType.DMA,
        ],
    )
    def k(x_hbm, pos_hbm, out_hbm, xb0, xb1, pb0, pb1,
          sem_x0, sem_x1, sem_p0, sem_p1, sem_s0, sem_s1):
        xbs = (xb0, xb1)
        pbs = (pb0, pb1)
        sem_x = (sem_x0, sem_x1)
        sem_p = (sem_p0, sem_p1)
        sem_s = (sem_s0, sem_s1)

        c = lax.axis_index("c")
        s = lax.axis_index("s")
        wid = s * _NC + c
        s_base = wid * s_per_w

        load_h = {}
        store_h = {}

        def issue_loads(it):
            slot = it % 2
            s0 = pl.multiple_of(s_base + it * SCHUNK, SCHUNK)
            hp = pltpu.async_copy(pos_hbm.at[pl.ds(s0, SCHUNK)],
                                  pbs[slot], sem_p[slot])
            hx = pltpu.async_copy(x_hbm.at[:, pl.ds(s0, SCHUNK)],
                                  xbs[slot], sem_x[slot])
            load_h[it] = (hp, hx)

        def wait_loads(it):
            hp, hx = load_h.pop(it)
            hp.wait()
            hx.wait()

        def compute(it):
            slot = it % 2
            xb = xbs[slot]
            pb = pbs[slot]

            def body(r, _):
                for j in range(D // _LANES):
                    sl = pl.ds(j * _LANES, _LANES)
                    v = pb[r, sl]
                    for b in range(B):
                        plsc.addupdate(xb.at[b, r, sl], v)
                return 0

            lax.fori_loop(0, SCHUNK, body, 0)

        def issue_store(it):
            slot = it % 2
            s0 = pl.multiple_of(s_base + it * SCHUNK, SCHUNK)
            store_h[it] = pltpu.async_copy(xbs[slot],
                                           out_hbm.at[:, pl.ds(s0, SCHUNK)],
                                           sem_s[slot])

        def wait_store(it):
            store_h.pop(it).wait()

        issue_loads(0)
        for it in range(n_iter):
            if it + 1 < n_iter:
                if it >= 1:
                    wait_store(it - 1)
                issue_loads(it + 1)
            wait_loads(it)
            compute(it)
            issue_store(it)
        wait_store(n_iter - 2)
        wait_store(n_iter - 1)

    return k


def kernel(x, pos_emb):
    B, S, D = x.shape
    k = _make_sc_add(B, S, D)
    return k(x, pos_emb)
